# Initial kernel scaffold; baseline (speedup 1.0000x reference)
import jax
import jax.numpy as jnp
from jax.experimental import pallas as pl


def kernel(x, Wg, W1, W2):
    raise NotImplementedError("write your pallas kernel here")


# speedup vs baseline: 1.5280x; 1.5280x over previous
import functools

import jax
import jax.numpy as jnp
from jax.experimental import pallas as pl
from jax.experimental.pallas import tpu as pltpu

T = 2048
D = 768
E = 8
FF = 3072
FFB = 768
NF = FF // FFB


def _moe_body(x_ref, wg_ref, w1_ref, w2_ref, y_ref, aux_ref, wfull_ref):
    e = pl.program_id(0)
    j = pl.program_id(1)

    @pl.when(jnp.logical_and(e == 0, j == 0))
    def _router():
        x = x_ref[...]
        logits = jax.lax.dot_general(
            x, wg_ref[...], (((1,), (1,)), ((), ())),
            preferred_element_type=jnp.float32)
        m = jnp.max(logits, axis=1, keepdims=True)
        ex = jnp.exp(logits - m)
        scores = ex / jnp.sum(ex, axis=1, keepdims=True)
        lane = jax.lax.broadcasted_iota(jnp.int32, (T, E), 1)
        m1 = jnp.max(scores, axis=1, keepdims=True)
        a1 = jnp.min(jnp.where(scores == m1, lane, E), axis=1, keepdims=True)
        s2 = jnp.where(lane == a1, -jnp.inf, scores)
        m2 = jnp.max(s2, axis=1, keepdims=True)
        a2 = jnp.min(jnp.where(s2 == m2, lane, E), axis=1, keepdims=True)
        denom = m1 + m2
        w1n = m1 / denom
        w2n = m2 / denom
        oh1 = (lane == a1).astype(jnp.float32)
        oh2 = (lane == a2).astype(jnp.float32)
        wfull_ref[...] = w1n * oh1 + w2n * oh2
        c1 = jnp.sum(oh1, axis=0)
        c2 = jnp.sum(oh2, axis=0)
        p1 = jnp.sum(oh1 * scores, axis=0)
        p2 = jnp.sum(oh2 * scores, axis=0)
        aux = (jnp.sum(p1 * c1) + jnp.sum(p2 * c2)) * (float(E) / float(T))
        aux_ref[0, 0] = aux

    h = jax.lax.dot_general(
        x_ref[...], w1_ref[0], (((1,), (1,)), ((), ())),
        preferred_element_type=jnp.float32)
    h = h * jax.lax.logistic(h)
    lane_e = jax.lax.broadcasted_iota(jnp.int32, (T, E), 1)
    w_e = jnp.sum(jnp.where(lane_e == e, wfull_ref[...], 0.0),
                  axis=1, keepdims=True)
    h = h * w_e
    contrib = jax.lax.dot_general(
        h, w2_ref[0], (((1,), (1,)), ((), ())),
        preferred_element_type=jnp.float32)

    @pl.when(jnp.logical_and(e == 0, j == 0))
    def _init():
        y_ref[...] = contrib

    @pl.when(jnp.logical_not(jnp.logical_and(e == 0, j == 0)))
    def _acc():
        y_ref[...] += contrib


@functools.partial(jax.jit, static_argnames=())
def kernel(x, Wg, W1, W2):
    b, s, d = x.shape
    flat = x.reshape(T, D)
    y, aux = pl.pallas_call(
        _moe_body,
        grid=(E, NF),
        in_specs=[
            pl.BlockSpec((T, D), lambda e, j: (0, 0)),
            pl.BlockSpec((E, D), lambda e, j: (0, 0)),
            pl.BlockSpec((1, FFB, D), lambda e, j: (e, j, 0)),
            pl.BlockSpec((1, D, FFB), lambda e, j: (e, 0, j)),
        ],
        out_specs=[
            pl.BlockSpec((T, D), lambda e, j: (0, 0)),
            pl.BlockSpec((1, 1), lambda e, j: (0, 0), memory_space=pltpu.SMEM),
        ],
        out_shape=[
            jax.ShapeDtypeStruct((T, D), jnp.float32),
            jax.ShapeDtypeStruct((1, 1), jnp.float32),
        ],
        scratch_shapes=[pltpu.VMEM((T, E), jnp.float32)],
        compiler_params=pltpu.CompilerParams(
            dimension_semantics=("arbitrary", "arbitrary"),
        ),
    )(flat, Wg, W1, W2)
    return y.reshape(b, s, d), aux.reshape(())
